# baseline (speedup 1.0000x reference)
import functools

import jax
import jax.numpy as jnp
from jax import lax
from jax.experimental import pallas as pl
from jax.experimental.pallas import tpu as pltpu
from jax.experimental.pallas import tpu_sc as plsc

HIDDEN = 128
BATCH = 16384
NUM_SG = 230
PAD_SG = 256

SC_ROWS = 8192
TC_ROWS = BATCH - SC_ROWS

NUM_CORES = 2
NUM_SUBCORES = 16
NW = NUM_CORES * NUM_SUBCORES
B_PER_W = SC_ROWS // NW
CHUNK = 64
N_CHUNKS = B_PER_W // CHUNK
LANES = 16

TC_BLOCK = 512
TC_GRID = TC_ROWS // TC_BLOCK


def _make_sc_kernel():
    mesh = plsc.VectorSubcoreMesh(core_axis_name="c", subcore_axis_name="s")

    @functools.partial(
        pl.kernel,
        mesh=mesh,
        out_type=jax.ShapeDtypeStruct((SC_ROWS, HIDDEN), jnp.float32),
        scratch_types=[
            pltpu.VMEM((N_CHUNKS, CHUNK), jnp.int32),
            pltpu.VMEM((B_PER_W, HIDDEN), jnp.float32),
            pltpu.VMEM_SHARED((NUM_SG, HIDDEN), jnp.float32),
        ]
        + [pltpu.SemaphoreType.DMA] * (N_CHUNKS + 1),
    )
    def k(x_hbm, table_hbm, out_hbm, idx_v, rows_v, table_sh, *sems):
        gather_sems, out_sem = sems[:N_CHUNKS], sems[N_CHUNKS]
        sid = lax.axis_index("s")
        wid = sid * NUM_CORES + lax.axis_index("c")
        base = wid * B_PER_W
        idx_cp = pltpu.async_copy(x_hbm.at[wid], idx_v, out_sem)

        @pl.when(sid == 0)
        def _():
            pltpu.sync_copy(table_hbm, table_sh)

        idx_cp.wait()
        for j in range(N_CHUNKS):
            for i in range(CHUNK // LANES):
                sl = pl.ds(i * LANES, LANES)
                idx_v[j, sl] = idx_v[j, sl] - 1
        plsc.subcore_barrier()
        gathers = []
        for j in range(N_CHUNKS):
            gathers.append(
                pltpu.async_copy(
                    table_sh.at[idx_v.at[j]],
                    rows_v.at[pl.ds(j * CHUNK, CHUNK)],
                    gather_sems[j],
                )
            )
        outs = []
        for j in range(N_CHUNKS):
            gathers[j].wait()
            outs.append(
                pltpu.async_copy(
                    rows_v.at[pl.ds(j * CHUNK, CHUNK)],
                    out_hbm.at[pl.ds(base + j * CHUNK, CHUNK)],
                    out_sem,
                )
            )
        for c in outs:
            c.wait()

    return k


_sc_lookup = _make_sc_kernel()


def _tc_body(idx_ref, tab_ref, out_ref):
    idx = idx_ref[0, 0, :] - 1
    iota = lax.broadcasted_iota(jnp.int32, (PAD_SG, TC_BLOCK), 0)
    one_hot = (iota == idx[None, :]).astype(jnp.float32)
    out_ref[...] = lax.dot_general(
        one_hot,
        tab_ref[...],
        (((0,), (0,)), ((), ())),
        preferred_element_type=jnp.float32,
    )


_tc_lookup = pl.pallas_call(
    _tc_body,
    grid=(TC_GRID,),
    in_specs=[
        pl.BlockSpec((1, 1, TC_BLOCK), lambda i: (i, 0, 0)),
        pl.BlockSpec((PAD_SG, HIDDEN), lambda i: (0, 0)),
    ],
    out_specs=pl.BlockSpec((TC_BLOCK, HIDDEN), lambda i: (i, 0)),
    out_shape=jax.ShapeDtypeStruct((TC_ROWS, HIDDEN), jnp.float32),
)


def kernel(x, table):
    x_sc = x[:SC_ROWS].reshape(NW, N_CHUNKS, CHUNK)
    x_tc = x[SC_ROWS:].reshape(TC_GRID, 1, TC_BLOCK)
    table_pad = jnp.zeros((PAD_SG, HIDDEN), jnp.float32).at[:NUM_SG].set(table)
    sc_out = _sc_lookup(x_sc, table)
    tc_out = _tc_lookup(x_tc, table_pad)
    return jnp.concatenate([sc_out, tc_out], axis=0)


# speedup vs baseline: 1.4704x; 1.4704x over previous
import functools

import jax
import jax.numpy as jnp
from jax import lax
from jax.experimental import pallas as pl
from jax.experimental.pallas import tpu as pltpu
from jax.experimental.pallas import tpu_sc as plsc

HIDDEN = 128
BATCH = 16384
NUM_SG = 230

NUM_CORES = 2
NUM_SUBCORES = 16
NW = NUM_CORES * NUM_SUBCORES
B_PER_W = BATCH // NW
CHUNK = 64
N_CHUNKS = B_PER_W // CHUNK


def _make_kernel():
    mesh = plsc.VectorSubcoreMesh(core_axis_name="c", subcore_axis_name="s")

    @functools.partial(
        pl.kernel,
        mesh=mesh,
        out_type=jax.ShapeDtypeStruct((BATCH, HIDDEN), jnp.float32),
        scratch_types=[
            pltpu.VMEM((N_CHUNKS, CHUNK), jnp.int32),
            pltpu.VMEM((B_PER_W, HIDDEN), jnp.float32),
            pltpu.VMEM_SHARED((NUM_SG + 1, HIDDEN), jnp.float32),
        ]
        + [pltpu.SemaphoreType.DMA] * (N_CHUNKS + 1),
    )
    def k(x_hbm, table_hbm, out_hbm, idx_v, rows_v, table_sh, *sems):
        gather_sems, out_sem = sems[:N_CHUNKS], sems[N_CHUNKS]
        sid = lax.axis_index("s")
        wid = sid * NUM_CORES + lax.axis_index("c")
        base = wid * B_PER_W
        idx_cp = pltpu.async_copy(x_hbm.at[wid], idx_v, out_sem)

        @pl.when(sid == 0)
        def _():
            pltpu.sync_copy(table_hbm, table_sh.at[pl.ds(1, NUM_SG)])

        idx_cp.wait()
        plsc.subcore_barrier()
        gathers = []
        for j in range(N_CHUNKS):
            gathers.append(
                pltpu.async_copy(
                    table_sh.at[idx_v.at[j]],
                    rows_v.at[pl.ds(j * CHUNK, CHUNK)],
                    gather_sems[j],
                )
            )
        outs = []
        for j in range(N_CHUNKS):
            gathers[j].wait()
            outs.append(
                pltpu.async_copy(
                    rows_v.at[pl.ds(j * CHUNK, CHUNK)],
                    out_hbm.at[pl.ds(base + j * CHUNK, CHUNK)],
                    out_sem,
                )
            )
        for c in outs:
            c.wait()

    return k


_sc_lookup = _make_kernel()


def kernel(x, table):
    idx3 = x.reshape(NW, N_CHUNKS, CHUNK)
    return _sc_lookup(idx3, table)
